# Initial kernel scaffold; baseline (speedup 1.0000x reference)
import jax
import jax.numpy as jnp
from jax.experimental import pallas as pl


def kernel(gene_id, gene_value, assay, suspension_type, gene_id_table, value_W, assay_table, susp_table):
    raise NotImplementedError("write your pallas kernel here")


# speedup vs baseline: 3.6677x; 3.6677x over previous
import functools

import jax
import jax.numpy as jnp
from jax import lax
from jax.experimental import pallas as pl
from jax.experimental.pallas import tpu as pltpu
from jax.experimental.pallas import tpu_sc as plsc

NC = 2
NS = 16
NW = NC * NS
D = 128
CHUNK = 128


def _sc_embed(gid, gvT, asy, ssp, table, comb, w, B):
    b_per_w = B // NW
    n_iter = b_per_w // CHUNK
    n_comb = comb.shape[0]

    mesh = plsc.VectorSubcoreMesh(
        core_axis_name="c", subcore_axis_name="s", num_cores=NC, num_subcores=NS
    )

    @functools.partial(
        pl.kernel,
        out_type=jax.ShapeDtypeStruct((B, D), jnp.float32),
        mesh=mesh,
        scratch_types=[
            pltpu.VMEM((CHUNK,), jnp.int32),
            pltpu.VMEM((CHUNK,), jnp.int32),
            pltpu.VMEM((CHUNK,), jnp.int32),
            pltpu.VMEM((3, CHUNK), jnp.float32),
            pltpu.VMEM((CHUNK, D), jnp.float32),
            pltpu.VMEM((n_comb, D), jnp.float32),
            pltpu.VMEM((3, D), jnp.float32),
            pltpu.SemaphoreType.DMA,
        ],
    )
    def body(gid_h, gv_h, asy_h, ssp_h, table_h, comb_h, w_h, out_h,
             idx_v, asy_v, ssp_v, gv_v, rows_v, comb_v, w_v, sem):
        wid = lax.axis_index("s") * NC + lax.axis_index("c")
        pltpu.sync_copy(comb_h, comb_v)
        pltpu.sync_copy(w_h, w_v)

        def step(g, carry):
            base = wid * b_per_w + g * CHUNK
            pltpu.sync_copy(gid_h.at[pl.ds(base, CHUNK)], idx_v)
            pltpu.sync_copy(asy_h.at[pl.ds(base, CHUNK)], asy_v)
            pltpu.sync_copy(ssp_h.at[pl.ds(base, CHUNK)], ssp_v)
            pltpu.sync_copy(gv_h.at[:, pl.ds(base, CHUNK)], gv_v)
            pltpu.async_copy(table_h.at[idx_v], rows_v, sem).wait()

            def group(grp, c2):
                tbase = grp * 16
                asy_vec = asy_v[pl.ds(tbase, 16)]
                ssp_vec = ssp_v[pl.ds(tbase, 16)]
                ci_vec = asy_vec * 5 + ssp_vec
                gv0 = gv_v[0, pl.ds(tbase, 16)]
                gv1 = gv_v[1, pl.ds(tbase, 16)]
                gv2 = gv_v[2, pl.ds(tbase, 16)]
                for j in range(16):
                    t = tbase + j
                    ci = ci_vec[j]
                    v0 = gv0[j]
                    v1 = gv1[j]
                    v2 = gv2[j]
                    for cc in range(D // 16):
                        off = cc * 16
                        acc = rows_v[t, pl.ds(off, 16)]
                        acc = acc + comb_v[ci, pl.ds(off, 16)]
                        acc = acc + v0 * w_v[0, pl.ds(off, 16)]
                        acc = acc + v1 * w_v[1, pl.ds(off, 16)]
                        acc = acc + v2 * w_v[2, pl.ds(off, 16)]
                        rows_v[t, pl.ds(off, 16)] = acc
                return c2

            lax.fori_loop(0, CHUNK // 16, group, 0)
            pltpu.sync_copy(rows_v, out_h.at[pl.ds(base, CHUNK), :])
            return carry

        lax.fori_loop(0, n_iter, step, 0)

    return body(gid, gvT, asy, ssp, table, comb, w)


def kernel(gene_id, gene_value, assay, suspension_type, gene_id_table,
           value_W, assay_table, susp_table):
    N, C = gene_id.shape
    B = N * C
    gid = gene_id.reshape(B).astype(jnp.int32)
    gvT = gene_value.reshape(B, 3).T
    asy = assay.reshape(B).astype(jnp.int32)
    ssp = suspension_type.reshape(B).astype(jnp.int32)
    comb = (assay_table[:, None, :] + susp_table[None, :, :]).reshape(-1, D)
    out = _sc_embed(gid, gvT, asy, ssp, gene_id_table, comb, value_W, B)
    return out.reshape(N, C, D)


# speedup vs baseline: 17.0482x; 4.6482x over previous
import functools

import jax
import jax.numpy as jnp
from jax import lax
from jax.experimental import pallas as pl
from jax.experimental.pallas import tpu as pltpu
from jax.experimental.pallas import tpu_sc as plsc

NC = 2
NS = 16
NW = NC * NS
D = 128
CHUNK = 128


def _sc_embed(gid, gvT, asy, ssp, table, comb, w, B):
    b_per_w = B // NW
    n_iter = b_per_w // CHUNK
    n_half = n_iter // 2
    n_comb = comb.shape[0]

    mesh = plsc.VectorSubcoreMesh(
        core_axis_name="c", subcore_axis_name="s", num_cores=NC, num_subcores=NS
    )

    @functools.partial(
        pl.kernel,
        out_type=jax.ShapeDtypeStruct((B, D), jnp.float32),
        mesh=mesh,
        scratch_types=[
            [pltpu.VMEM((CHUNK,), jnp.int32)] * 2,
            [pltpu.VMEM((CHUNK,), jnp.int32)] * 2,
            [pltpu.VMEM((CHUNK,), jnp.int32)] * 2,
            [pltpu.VMEM((3, CHUNK), jnp.float32)] * 2,
            [pltpu.VMEM((CHUNK, D), jnp.float32)] * 2,
            [pltpu.VMEM((CHUNK, D), jnp.float32)] * 2,
            pltpu.VMEM((n_comb, D), jnp.float32),
            pltpu.VMEM((3, D), jnp.float32),
            [pltpu.SemaphoreType.DMA] * 2,
            [pltpu.SemaphoreType.DMA] * 2,
            [pltpu.SemaphoreType.DMA] * 2,
        ],
    )
    def body(gid_h, gv_h, asy_h, ssp_h, table_h, comb_h, w_h, out_h,
             idx_v, asy_v, ssp_v, gv_v, rows_v, res_v, comb_v, w_v,
             ssem, gsem, osem):
        wid = lax.axis_index("s") * NC + lax.axis_index("c")
        pltpu.sync_copy(comb_h, comb_v)
        pltpu.sync_copy(w_h, w_v)
        wvec = [[w_v[k, pl.ds(cc * 16, 16)] for cc in range(D // 16)]
                for k in range(3)]
        w0 = wid * b_per_w

        def issue_smalls(c, b):
            base = w0 + c * CHUNK
            pltpu.async_copy(gid_h.at[pl.ds(base, CHUNK)], idx_v[b], ssem[b])
            pltpu.async_copy(asy_h.at[pl.ds(base, CHUNK)], asy_v[b], ssem[b])
            pltpu.async_copy(ssp_h.at[pl.ds(base, CHUNK)], ssp_v[b], ssem[b])
            pltpu.async_copy(gv_h.at[:, pl.ds(base, CHUNK)], gv_v[b], ssem[b])

        def wait_smalls(b):
            base = 0
            pltpu.make_async_copy(gid_h.at[pl.ds(base, CHUNK)], idx_v[b], ssem[b]).wait()
            pltpu.make_async_copy(asy_h.at[pl.ds(base, CHUNK)], asy_v[b], ssem[b]).wait()
            pltpu.make_async_copy(ssp_h.at[pl.ds(base, CHUNK)], ssp_v[b], ssem[b]).wait()
            pltpu.make_async_copy(gv_h.at[:, pl.ds(base, CHUNK)], gv_v[b], ssem[b]).wait()

        def issue_gather(b):
            pltpu.async_copy(table_h.at[idx_v[b]], rows_v[b], gsem[b])

        def wait_gather(b):
            pltpu.make_async_copy(table_h.at[idx_v[b]], rows_v[b], gsem[b]).wait()

        def issue_out(c, b):
            base = w0 + c * CHUNK
            pltpu.async_copy(res_v[b], out_h.at[pl.ds(base, CHUNK), :], osem[b])

        def wait_out(b):
            pltpu.make_async_copy(res_v[b], out_h.at[pl.ds(0, CHUNK), :], osem[b]).wait()

        def compute(b):
            def group(grp, c2):
                tbase = grp * 16
                asy_vec = asy_v[b][pl.ds(tbase, 16)]
                ssp_vec = ssp_v[b][pl.ds(tbase, 16)]
                ci_vec = asy_vec * 5 + ssp_vec
                gv0 = gv_v[b][0, pl.ds(tbase, 16)]
                gv1 = gv_v[b][1, pl.ds(tbase, 16)]
                gv2 = gv_v[b][2, pl.ds(tbase, 16)]
                nc = D // 16
                for j in range(16):
                    t = tbase + j
                    ci = ci_vec[j]
                    v0 = gv0[j]
                    v1 = gv1[j]
                    v2 = gv2[j]
                    rr = [rows_v[b][t, pl.ds(cc * 16, 16)] for cc in range(nc)]
                    cb = [comb_v[ci, pl.ds(cc * 16, 16)] for cc in range(nc)]
                    dn = [(v0 * wvec[0][cc] + v1 * wvec[1][cc])
                          + v2 * wvec[2][cc] for cc in range(nc)]
                    for cc in range(nc):
                        res_v[b][t, pl.ds(cc * 16, 16)] = (rr[cc] + cb[cc]) + dn[cc]
                return c2

            lax.fori_loop(0, CHUNK // 16, group, 0)

        issue_smalls(0, 0)
        issue_smalls(1, 1)
        wait_smalls(0)
        issue_gather(0)

        def half_step(i, carry):
            for b in (0, 1):
                c = i * 2 + b
                wait_gather(b)

                bb = 1 - b
                if b == 0:
                    wait_smalls(bb)
                    issue_gather(bb)
                else:
                    @pl.when(i < n_half - 1)
                    def _():
                        wait_smalls(bb)
                        issue_gather(bb)

                @pl.when(i >= 1)
                def _():
                    wait_out(b)

                compute(b)
                issue_out(c, b)

                @pl.when(i < n_half - 1)
                def _():
                    issue_smalls(c + 2, b)
            return carry

        lax.fori_loop(0, n_half, half_step, 0)
        wait_out(0)
        wait_out(1)

    return body(gid, gvT, asy, ssp, table, comb, w)


def kernel(gene_id, gene_value, assay, suspension_type, gene_id_table,
           value_W, assay_table, susp_table):
    N, C = gene_id.shape
    B = N * C
    gid = gene_id.reshape(B).astype(jnp.int32)
    gvT = gene_value.reshape(B, 3).T
    asy = assay.reshape(B).astype(jnp.int32)
    ssp = suspension_type.reshape(B).astype(jnp.int32)
    comb = (assay_table[:, None, :] + susp_table[None, :, :]).reshape(-1, D)
    out = _sc_embed(gid, gvT, asy, ssp, gene_id_table, comb, value_W, B)
    return out.reshape(N, C, D)
